# baseline (speedup 1.0000x reference)
import functools

import jax
import jax.numpy as jnp
from jax import lax
from jax.experimental import pallas as pl
from jax.experimental.pallas import tpu as pltpu
from jax.experimental.pallas import tpu_sc as plsc

_NUM_CORES = 2
_NUM_SUBCORES = 16
_NUM_WORKERS = _NUM_CORES * _NUM_SUBCORES


def _gather_call(table, idx):
    B = idx.shape[0]
    D = table.shape[1]
    b_per_w = B // _NUM_WORKERS
    mesh = plsc.VectorSubcoreMesh(core_axis_name="c", subcore_axis_name="s")

    @functools.partial(
        pl.kernel,
        mesh=mesh,
        out_type=jax.ShapeDtypeStruct((B, D), jnp.float32),
        scratch_types=[
            pltpu.VMEM((b_per_w,), jnp.int32),
            pltpu.VMEM((b_per_w, D), jnp.float32),
            pltpu.SemaphoreType.DMA,
        ],
    )
    def gather_kernel(table_hbm, idx_hbm, out_hbm, idx_v, rows_v, gsem):
        wid = lax.axis_index("s") * _NUM_CORES + lax.axis_index("c")
        base = wid * b_per_w
        pltpu.sync_copy(idx_hbm.at[pl.ds(base, b_per_w)], idx_v)
        pltpu.async_copy(table_hbm.at[idx_v], rows_v, gsem).wait()
        pltpu.sync_copy(rows_v, out_hbm.at[pl.ds(base, b_per_w)])

    return gather_kernel(table, idx)


_TC_ROWS_PER_BLOCK = 1024


def _sin_call(t_col, div2, phase):
    B_tc = t_col.shape[0]
    D = div2.shape[1]
    R = min(_TC_ROWS_PER_BLOCK, B_tc)

    def body(t_ref, div_ref, ph_ref, o_ref):
        o_ref[:, :] = jnp.sin(t_ref[:, :] / div_ref[:, :] + ph_ref[:, :])

    return pl.pallas_call(
        body,
        grid=(B_tc // R,),
        in_specs=[
            pl.BlockSpec((R, 1), lambda i: (i, 0)),
            pl.BlockSpec((1, D), lambda i: (0, 0)),
            pl.BlockSpec((1, D), lambda i: (0, 0)),
        ],
        out_specs=pl.BlockSpec((R, D), lambda i: (i, 0)),
        out_shape=jax.ShapeDtypeStruct((B_tc, D), jnp.float32),
    )(t_col, div2, phase)


_SC_FRACTION_NUM = 1
_SC_FRACTION_DEN = 2


def kernel(x, t, embeddings):
    B = t.shape[0]
    D = embeddings.shape[1]
    t = t.astype(jnp.int32)
    B_sc = (B * _SC_FRACTION_NUM // _SC_FRACTION_DEN) // _NUM_WORKERS * _NUM_WORKERS
    sc_out = _gather_call(embeddings, t[:B_sc])
    k = jnp.arange(0, D, 2, dtype=jnp.float32)
    divisor = jnp.asarray(10000.0, dtype=jnp.float32) ** (k / D)
    div2 = jnp.repeat(divisor, 2)[None, :]
    phase = jnp.where(jnp.arange(D) % 2 == 1, jnp.float32(jnp.pi / 2), 0.0)[None, :]
    t_col = t[B_sc:].astype(jnp.float32)[:, None]
    tc_out = _sin_call(t_col, div2, phase)
    out = jnp.concatenate([sc_out, tc_out], axis=0)
    return out[:, :, None, None]


# speedup vs baseline: 1.5628x; 1.5628x over previous
import functools

import jax
import jax.numpy as jnp
from jax import lax
from jax.experimental import pallas as pl
from jax.experimental.pallas import tpu as pltpu
from jax.experimental.pallas import tpu_sc as plsc

_NUM_CORES = 2
_NUM_SUBCORES = 16
_NUM_WORKERS = _NUM_CORES * _NUM_SUBCORES


def _gather_call(table, idx):
    B = idx.shape[0]
    D = table.shape[1]
    b_per_w = B // _NUM_WORKERS
    mesh = plsc.VectorSubcoreMesh(core_axis_name="c", subcore_axis_name="s")

    @functools.partial(
        pl.kernel,
        mesh=mesh,
        out_type=jax.ShapeDtypeStruct((B, D), jnp.float32),
        scratch_types=[
            pltpu.VMEM((b_per_w,), jnp.int32),
            pltpu.VMEM((b_per_w, D), jnp.float32),
            pltpu.SemaphoreType.DMA,
        ],
    )
    def gather_kernel(table_hbm, idx_hbm, out_hbm, idx_v, rows_v, gsem):
        wid = lax.axis_index("s") * _NUM_CORES + lax.axis_index("c")
        base = wid * b_per_w
        pltpu.sync_copy(idx_hbm.at[pl.ds(base, b_per_w)], idx_v)
        pltpu.async_copy(table_hbm.at[idx_v], rows_v, gsem).wait()
        pltpu.sync_copy(rows_v, out_hbm.at[pl.ds(base, b_per_w)])

    return gather_kernel(table, idx)


def kernel(x, t, embeddings):
    out = _gather_call(embeddings, t.astype(jnp.int32))
    return out[:, :, None, None]
